# Initial kernel scaffold; baseline (speedup 1.0000x reference)
import jax
import jax.numpy as jnp
from jax.experimental import pallas as pl


def kernel(x, W):
    raise NotImplementedError("write your pallas kernel here")


# speedup vs baseline: 1.1542x; 1.1542x over previous
import jax
import jax.numpy as jnp
from jax.experimental import pallas as pl
from jax.experimental.pallas import tpu as pltpu

CB = 4
KV = 1024
DIM = 256
NT = 16384
BT = 256
NB = NT // BT
CW = 0.25
NPAIRS = KV * (KV - 1) // 2


def _rvq_body(x_ref, w_ref, q_out, idx_out, scal_out, counts_ref, acc_ref):
    b = pl.program_id(0)

    @pl.when(b == 0)
    def _init():
        counts_ref[...] = jnp.zeros_like(counts_ref)
        for i in range(CB):
            acc_ref[i] = jnp.float32(0.0)

    xb = x_ref[...]
    quant = jnp.zeros_like(xb)
    r = xb
    iota_k = jax.lax.broadcasted_iota(jnp.int32, (BT, KV), 1)
    for i in range(CB):
        w = w_ref[i]
        sw = jnp.sum(w * w, axis=1)
        sx = jnp.sum(r * r, axis=1, keepdims=True)
        m = jax.lax.dot_general(r, w, (((1,), (1,)), ((), ())),
                                preferred_element_type=jnp.float32)
        d = (sx + sw[None, :]) - 2.0 * m
        dmin = jnp.min(d, axis=1, keepdims=True)
        idx = jnp.min(jnp.where(d == dmin, iota_k, KV), axis=1)
        onehot = (iota_k == idx[:, None]).astype(jnp.float32)
        q = jax.lax.dot_general(onehot, w, (((1,), (0,)), ((), ())),
                                preferred_element_type=jnp.float32)
        diff = q - r
        acc_ref[i] += jnp.sum(diff * diff)
        counts_ref[pl.ds(i, 1), :] += jnp.sum(onehot, axis=0, keepdims=True)
        cur = r + (q - r)
        quant = quant + cur
        r = xb - quant
        idx_out[0, i, :] = idx
    q_out[...] = quant

    @pl.when(b == NB - 1)
    def _finalize():
        total_quant = jnp.float32(0.0)
        for i in range(CB):
            mloss = acc_ref[i] / jnp.float32(NT * DIM)
            total_quant = total_quant + (mloss + CW * mloss)
        total_util = jnp.float32(0.0)
        for i in range(CB):
            c = counts_ref[pl.ds(i, 1), :]
            total_util = total_util + jnp.mean(jnp.abs(c - jnp.float32(NT / KV)))
        total_compact = jnp.float32(0.0)
        rowi = jax.lax.broadcasted_iota(jnp.int32, (KV, KV), 0)
        coli = jax.lax.broadcasted_iota(jnp.int32, (KV, KV), 1)
        mask = coli > rowi
        for i in range(CB):
            w = w_ref[i]
            sq = jnp.sum(w * w, axis=1)
            m2 = jax.lax.dot_general(w, w, (((1,), (1,)), ((), ())),
                                     preferred_element_type=jnp.float32)
            d2 = (sq[:, None] + sq[None, :]) - 2.0 * m2
            dd = jnp.sqrt(jnp.maximum(d2, 1e-12))
            total_compact = total_compact + 2.0 * (
                jnp.sum(jnp.where(mask, dd, 0.0)) / jnp.float32(NPAIRS))
        lane = jax.lax.broadcasted_iota(jnp.int32, (8, 128), 1)
        sub = jax.lax.broadcasted_iota(jnp.int32, (8, 128), 0)
        out = jnp.where((sub == 0) & (lane == 0), total_quant, 0.0)
        out = jnp.where((sub == 0) & (lane == 1), total_util, out)
        out = jnp.where((sub == 0) & (lane == 2), total_compact, out)
        scal_out[...] = out


def kernel(x, W):
    q, idx3, scal = pl.pallas_call(
        _rvq_body,
        grid=(NB,),
        in_specs=[
            pl.BlockSpec((BT, DIM), lambda b: (b, 0)),
            pl.BlockSpec((CB, KV, DIM), lambda b: (0, 0, 0)),
        ],
        out_specs=[
            pl.BlockSpec((BT, DIM), lambda b: (b, 0)),
            pl.BlockSpec((1, CB, BT), lambda b: (b, 0, 0)),
            pl.BlockSpec((8, 128), lambda b: (0, 0)),
        ],
        out_shape=[
            jax.ShapeDtypeStruct((NT, DIM), jnp.float32),
            jax.ShapeDtypeStruct((NB, CB, BT), jnp.int32),
            jax.ShapeDtypeStruct((8, 128), jnp.float32),
        ],
        scratch_shapes=[
            pltpu.VMEM((CB, KV), jnp.float32),
            pltpu.SMEM((CB,), jnp.float32),
        ],
    )(x, W)
    indices = jnp.transpose(idx3, (1, 0, 2)).reshape(CB, NT)
    return (q, scal[0, 0].reshape(()), scal[0, 1].reshape(()),
            scal[0, 2].reshape(()), indices)


# speedup vs baseline: 1.3659x; 1.1834x over previous
import jax
import jax.numpy as jnp
from jax.experimental import pallas as pl
from jax.experimental.pallas import tpu as pltpu

CB = 4
KV = 1024
DIM = 256
NT = 16384
BT = 256
NB = NT // BT
CW = 0.25
NPAIRS = KV * (KV - 1) // 2


def _rvq_body(x_ref, w_ref, q_out, idx_out, scal_out, counts_ref, acc_ref,
              sw_ref):
    b = pl.program_id(0)

    @pl.when(b == 0)
    def _init():
        counts_ref[...] = jnp.zeros_like(counts_ref)
        for i in range(CB):
            acc_ref[i] = jnp.float32(0.0)
            w = w_ref[i]
            sw_ref[pl.ds(i, 1), :] = jnp.sum(w * w, axis=1).reshape(1, KV)

    xb = x_ref[...]
    quant = jnp.zeros_like(xb)
    r = xb
    iota_kf = jax.lax.broadcasted_iota(jnp.int32, (BT, KV), 1).astype(jnp.float32)
    ones_row = jnp.ones((8, BT), jnp.float32)
    for i in range(CB):
        w = w_ref[i]
        sw = sw_ref[pl.ds(i, 1), :]
        sx = jnp.sum(r * r, axis=1, keepdims=True)
        m2 = jax.lax.dot_general(r + r, w, (((1,), (1,)), ((), ())),
                                 preferred_element_type=jnp.float32)
        d = (sx + sw) - m2
        dmin = jnp.min(d, axis=1, keepdims=True)
        idxf = jnp.min(jnp.where(d == dmin, iota_kf, 8192.0), axis=1,
                       keepdims=True)
        onehot = (iota_kf == idxf).astype(jnp.float32)
        q = jax.lax.dot_general(onehot, w, (((1,), (0,)), ((), ())),
                                preferred_element_type=jnp.float32)
        diff = q - r
        acc_ref[i] += jnp.sum(diff * diff)
        cnt = jax.lax.dot_general(ones_row, onehot, (((1,), (0,)), ((), ())),
                                  preferred_element_type=jnp.float32)
        counts_ref[pl.ds(i, 1), :] += cnt[0:1, :]
        cur = r + (q - r)
        quant = quant + cur
        r = xb - quant
        idx_out[:, i:i + 1] = idxf.astype(jnp.int32)
    q_out[...] = quant

    @pl.when(b == NB - 1)
    def _finalize():
        total_quant = jnp.float32(0.0)
        for i in range(CB):
            mloss = acc_ref[i] / jnp.float32(NT * DIM)
            total_quant = total_quant + (mloss + CW * mloss)
        total_util = jnp.float32(0.0)
        for i in range(CB):
            c = counts_ref[pl.ds(i, 1), :]
            total_util = total_util + jnp.mean(jnp.abs(c - jnp.float32(NT / KV)))
        total_compact = jnp.float32(0.0)
        rowi = jax.lax.broadcasted_iota(jnp.int32, (KV, KV), 0)
        coli = jax.lax.broadcasted_iota(jnp.int32, (KV, KV), 1)
        mask = coli > rowi
        for i in range(CB):
            w = w_ref[i]
            sq = jnp.sum(w * w, axis=1)
            m2 = jax.lax.dot_general(w, w, (((1,), (1,)), ((), ())),
                                     preferred_element_type=jnp.float32)
            d2 = (sq[:, None] + sq[None, :]) - 2.0 * m2
            dd = jnp.sqrt(jnp.maximum(d2, 1e-12))
            total_compact = total_compact + 2.0 * (
                jnp.sum(jnp.where(mask, dd, 0.0)) / jnp.float32(NPAIRS))
        lane = jax.lax.broadcasted_iota(jnp.int32, (8, 128), 1)
        sub = jax.lax.broadcasted_iota(jnp.int32, (8, 128), 0)
        out = jnp.where((sub == 0) & (lane == 0), total_quant, 0.0)
        out = jnp.where((sub == 0) & (lane == 1), total_util, out)
        out = jnp.where((sub == 0) & (lane == 2), total_compact, out)
        scal_out[...] = out


def kernel(x, W):
    q, idx3, scal = pl.pallas_call(
        _rvq_body,
        grid=(NB,),
        in_specs=[
            pl.BlockSpec((BT, DIM), lambda b: (b, 0)),
            pl.BlockSpec((CB, KV, DIM), lambda b: (0, 0, 0)),
        ],
        out_specs=[
            pl.BlockSpec((BT, DIM), lambda b: (b, 0)),
            pl.BlockSpec((BT, CB), lambda b: (b, 0)),
            pl.BlockSpec((8, 128), lambda b: (0, 0)),
        ],
        out_shape=[
            jax.ShapeDtypeStruct((NT, DIM), jnp.float32),
            jax.ShapeDtypeStruct((NT, CB), jnp.int32),
            jax.ShapeDtypeStruct((8, 128), jnp.float32),
        ],
        scratch_shapes=[
            pltpu.VMEM((CB, KV), jnp.float32),
            pltpu.SMEM((CB,), jnp.float32),
            pltpu.VMEM((CB, KV), jnp.float32),
        ],
    )(x, W)
    indices = jnp.transpose(idx3, (1, 0))
    return (q, scal[0, 0].reshape(()), scal[0, 1].reshape(()),
            scal[0, 2].reshape(()), indices)


# speedup vs baseline: 1.9212x; 1.4066x over previous
import jax
import jax.numpy as jnp
from jax.experimental import pallas as pl
from jax.experimental.pallas import tpu as pltpu

CB = 4
KV = 1024
DIM = 256
NT = 16384
BT = 1024
NB = NT // BT
CW = 0.25
NPAIRS = KV * (KV - 1) // 2


def _rvq_body(x_ref, w_ref, q_out, idx_out, scal_out, counts_ref, acc_ref,
              sw_ref):
    b = pl.program_id(0)

    @pl.when(b == 0)
    def _init():
        counts_ref[...] = jnp.zeros_like(counts_ref)
        for i in range(CB):
            acc_ref[i] = jnp.float32(0.0)
            w = w_ref[i]
            sw_ref[pl.ds(i, 1), :] = jnp.sum(w * w, axis=1).reshape(1, KV)

    xb = x_ref[...]
    quant = jnp.zeros_like(xb)
    r = xb
    iota_kf = jax.lax.broadcasted_iota(jnp.int32, (BT, KV), 1).astype(jnp.float32)
    ones_row = jnp.ones((8, BT), jnp.float32)
    for i in range(CB):
        w = w_ref[i]
        sw = sw_ref[pl.ds(i, 1), :]
        sx = jnp.sum(r * r, axis=1, keepdims=True)
        m2 = jax.lax.dot_general(r + r, w, (((1,), (1,)), ((), ())),
                                 preferred_element_type=jnp.float32)
        d = (sx + sw) - m2
        dmin = jnp.min(d, axis=1, keepdims=True)
        idxf = jnp.min(jnp.where(d == dmin, iota_kf, 8192.0), axis=1,
                       keepdims=True)
        onehot = (iota_kf == idxf).astype(jnp.float32)
        q = jax.lax.dot_general(onehot, w, (((1,), (0,)), ((), ())),
                                preferred_element_type=jnp.float32)
        diff = q - r
        acc_ref[i] += jnp.sum(diff * diff)
        cnt = jax.lax.dot_general(ones_row, onehot, (((1,), (0,)), ((), ())),
                                  preferred_element_type=jnp.float32)
        counts_ref[pl.ds(i, 1), :] += cnt[0:1, :]
        cur = r + (q - r)
        quant = quant + cur
        r = xb - quant
        idx_out[:, i:i + 1] = idxf.astype(jnp.int32)
    q_out[...] = quant

    @pl.when(b == NB - 1)
    def _finalize():
        total_quant = jnp.float32(0.0)
        for i in range(CB):
            mloss = acc_ref[i] / jnp.float32(NT * DIM)
            total_quant = total_quant + (mloss + CW * mloss)
        total_util = jnp.float32(0.0)
        for i in range(CB):
            c = counts_ref[pl.ds(i, 1), :]
            total_util = total_util + jnp.mean(jnp.abs(c - jnp.float32(NT / KV)))
        total_compact = jnp.float32(0.0)
        rowi = jax.lax.broadcasted_iota(jnp.int32, (KV, KV), 0)
        coli = jax.lax.broadcasted_iota(jnp.int32, (KV, KV), 1)
        mask = coli > rowi
        for i in range(CB):
            w = w_ref[i]
            sq = jnp.sum(w * w, axis=1)
            m2 = jax.lax.dot_general(w, w, (((1,), (1,)), ((), ())),
                                     preferred_element_type=jnp.float32)
            d2 = (sq[:, None] + sq[None, :]) - 2.0 * m2
            dd = jnp.sqrt(jnp.maximum(d2, 1e-12))
            total_compact = total_compact + 2.0 * (
                jnp.sum(jnp.where(mask, dd, 0.0)) / jnp.float32(NPAIRS))
        lane = jax.lax.broadcasted_iota(jnp.int32, (8, 128), 1)
        sub = jax.lax.broadcasted_iota(jnp.int32, (8, 128), 0)
        out = jnp.where((sub == 0) & (lane == 0), total_quant, 0.0)
        out = jnp.where((sub == 0) & (lane == 1), total_util, out)
        out = jnp.where((sub == 0) & (lane == 2), total_compact, out)
        scal_out[...] = out


def kernel(x, W):
    q, idx3, scal = pl.pallas_call(
        _rvq_body,
        grid=(NB,),
        in_specs=[
            pl.BlockSpec((BT, DIM), lambda b: (b, 0)),
            pl.BlockSpec((CB, KV, DIM), lambda b: (0, 0, 0)),
        ],
        out_specs=[
            pl.BlockSpec((BT, DIM), lambda b: (b, 0)),
            pl.BlockSpec((BT, CB), lambda b: (b, 0)),
            pl.BlockSpec((8, 128), lambda b: (0, 0)),
        ],
        out_shape=[
            jax.ShapeDtypeStruct((NT, DIM), jnp.float32),
            jax.ShapeDtypeStruct((NT, CB), jnp.int32),
            jax.ShapeDtypeStruct((8, 128), jnp.float32),
        ],
        scratch_shapes=[
            pltpu.VMEM((CB, KV), jnp.float32),
            pltpu.SMEM((CB,), jnp.float32),
            pltpu.VMEM((CB, KV), jnp.float32),
        ],
    )(x, W)
    indices = jnp.transpose(idx3, (1, 0))
    return (q, scal[0, 0].reshape(()), scal[0, 1].reshape(()),
            scal[0, 2].reshape(()), indices)


# speedup vs baseline: 2.0995x; 1.0928x over previous
import jax
import jax.numpy as jnp
from jax.experimental import pallas as pl
from jax.experimental.pallas import tpu as pltpu

CB = 4
KV = 1024
DIM = 256
NT = 16384
BT = 1024
NB = NT // BT
CW = 0.25
NPAIRS = KV * (KV - 1) // 2


def _rvq_body(x_ref, w_ref, q_out, idx_out, scal_out, counts_ref, acc_ref,
              sw_ref):
    b = pl.program_id(0)

    @pl.when(b == 0)
    def _init():
        counts_ref[...] = jnp.zeros_like(counts_ref)
        for i in range(CB):
            acc_ref[i] = jnp.float32(0.0)
            w = w_ref[i]
            sw_ref[pl.ds(i, 1), :] = jnp.sum(w * w, axis=1).reshape(1, KV)

    xb = x_ref[...]
    quant = jnp.zeros_like(xb)
    r = xb
    iota_kf = jax.lax.broadcasted_iota(jnp.int32, (BT, KV), 1).astype(jnp.float32)
    lane_f = jax.lax.broadcasted_iota(jnp.int32, (BT, 128), 1).astype(jnp.float32)
    ones_row = jnp.ones((8, BT), jnp.float32)
    NCH = KV // 128
    for i in range(CB):
        w = w_ref[i]
        sw = sw_ref[pl.ds(i, 1), :]
        sx = jnp.sum(r * r, axis=1, keepdims=True)
        m2 = jax.lax.dot_general(r + r, w, (((1,), (1,)), ((), ())),
                                 preferred_element_type=jnp.float32)
        acc_v = (sx + sw[:, 0:128]) - m2[:, 0:128]
        acc_c = jnp.zeros((BT, 128), jnp.float32)
        for c in range(1, NCH):
            dc = (sx + sw[:, c * 128:(c + 1) * 128]) - m2[:, c * 128:(c + 1) * 128]
            lt = dc < acc_v
            acc_c = jnp.where(lt, jnp.float32(c), acc_c)
            acc_v = jnp.minimum(dc, acc_v)
        dmin = jnp.min(acc_v, axis=1, keepdims=True)
        key = jnp.where(acc_v == dmin, acc_c * 128.0 + lane_f, 8192.0)
        idxf = jnp.min(key, axis=1, keepdims=True)
        onehot = (iota_kf == idxf).astype(jnp.float32)
        q = jax.lax.dot_general(onehot, w, (((1,), (0,)), ((), ())),
                                preferred_element_type=jnp.float32)
        diff = q - r
        acc_ref[i] += jnp.sum(diff * diff)
        cnt = jax.lax.dot_general(ones_row, onehot, (((1,), (0,)), ((), ())),
                                  preferred_element_type=jnp.float32)
        counts_ref[pl.ds(i, 1), :] += cnt[0:1, :]
        cur = r + (q - r)
        quant = quant + cur
        r = xb - quant
        idx_out[:, i:i + 1] = idxf.astype(jnp.int32)
    q_out[...] = quant

    @pl.when(b == NB - 1)
    def _finalize():
        total_quant = jnp.float32(0.0)
        for i in range(CB):
            mloss = acc_ref[i] / jnp.float32(NT * DIM)
            total_quant = total_quant + (mloss + CW * mloss)
        total_util = jnp.float32(0.0)
        for i in range(CB):
            c = counts_ref[pl.ds(i, 1), :]
            total_util = total_util + jnp.mean(jnp.abs(c - jnp.float32(NT / KV)))
        total_compact = jnp.float32(0.0)
        rowi = jax.lax.broadcasted_iota(jnp.int32, (KV, KV), 0)
        coli = jax.lax.broadcasted_iota(jnp.int32, (KV, KV), 1)
        mask = coli > rowi
        for i in range(CB):
            w = w_ref[i]
            sq = jnp.sum(w * w, axis=1)
            m2 = jax.lax.dot_general(w, w, (((1,), (1,)), ((), ())),
                                     preferred_element_type=jnp.float32)
            d2 = (sq[:, None] + sq[None, :]) - 2.0 * m2
            dd = jnp.sqrt(jnp.maximum(d2, 1e-12))
            total_compact = total_compact + 2.0 * (
                jnp.sum(jnp.where(mask, dd, 0.0)) / jnp.float32(NPAIRS))
        lane = jax.lax.broadcasted_iota(jnp.int32, (8, 128), 1)
        sub = jax.lax.broadcasted_iota(jnp.int32, (8, 128), 0)
        out = jnp.where((sub == 0) & (lane == 0), total_quant, 0.0)
        out = jnp.where((sub == 0) & (lane == 1), total_util, out)
        out = jnp.where((sub == 0) & (lane == 2), total_compact, out)
        scal_out[...] = out


def kernel(x, W):
    q, idx3, scal = pl.pallas_call(
        _rvq_body,
        grid=(NB,),
        in_specs=[
            pl.BlockSpec((BT, DIM), lambda b: (b, 0)),
            pl.BlockSpec((CB, KV, DIM), lambda b: (0, 0, 0)),
        ],
        out_specs=[
            pl.BlockSpec((BT, DIM), lambda b: (b, 0)),
            pl.BlockSpec((BT, CB), lambda b: (b, 0)),
            pl.BlockSpec((8, 128), lambda b: (0, 0)),
        ],
        out_shape=[
            jax.ShapeDtypeStruct((NT, DIM), jnp.float32),
            jax.ShapeDtypeStruct((NT, CB), jnp.int32),
            jax.ShapeDtypeStruct((8, 128), jnp.float32),
        ],
        scratch_shapes=[
            pltpu.VMEM((CB, KV), jnp.float32),
            pltpu.SMEM((CB,), jnp.float32),
            pltpu.VMEM((CB, KV), jnp.float32),
        ],
    )(x, W)
    indices = jnp.transpose(idx3, (1, 0))
    return (q, scal[0, 0].reshape(()), scal[0, 1].reshape(()),
            scal[0, 2].reshape(()), indices)


# speedup vs baseline: 2.2533x; 1.0733x over previous
import jax
import jax.numpy as jnp
from jax.experimental import pallas as pl
from jax.experimental.pallas import tpu as pltpu

CB = 4
KV = 1024
DIM = 256
NT = 16384
BT = 2048
NB = NT // BT
CW = 0.25
NPAIRS = KV * (KV - 1) // 2


def _rvq_body(x_ref, w_ref, q_out, idx_out, scal_out, counts_ref, acc_ref,
              sw_ref):
    b = pl.program_id(0)

    @pl.when(b == 0)
    def _init():
        counts_ref[...] = jnp.zeros_like(counts_ref)
        for i in range(CB):
            acc_ref[i] = jnp.float32(0.0)
            w = w_ref[i]
            sw_ref[pl.ds(i, 1), :] = jnp.sum(w * w, axis=1).reshape(1, KV)

    xb = x_ref[...]
    quant = jnp.zeros_like(xb)
    r = xb
    iota_ki = jax.lax.broadcasted_iota(jnp.int32, (BT, KV), 1)
    lane_f = jax.lax.broadcasted_iota(jnp.int32, (BT, 128), 1).astype(jnp.float32)
    ones_row = jnp.ones((8, BT), jnp.float32)
    NCH = KV // 128
    for i in range(CB):
        w = w_ref[i]
        sw = sw_ref[pl.ds(i, 1), :]
        sx = jnp.sum(r * r, axis=1, keepdims=True)
        m2 = jax.lax.dot_general(r + r, w, (((1,), (1,)), ((), ())),
                                 preferred_element_type=jnp.float32)
        acc_v = (sx + sw[:, 0:128]) - m2[:, 0:128]
        acc_c = jnp.zeros((BT, 128), jnp.float32)
        for c in range(1, NCH):
            dc = (sx + sw[:, c * 128:(c + 1) * 128]) - m2[:, c * 128:(c + 1) * 128]
            lt = dc < acc_v
            acc_c = jnp.where(lt, jnp.float32(c), acc_c)
            acc_v = jnp.minimum(dc, acc_v)
        dmin = jnp.min(acc_v, axis=1, keepdims=True)
        key = jnp.where(acc_v == dmin, acc_c * 128.0 + lane_f, 8192.0)
        idxf = jnp.min(key, axis=1, keepdims=True)
        idxi = idxf.astype(jnp.int32)
        onehot = (iota_ki == idxi).astype(jnp.float32)
        q = jax.lax.dot_general(onehot, w, (((1,), (0,)), ((), ())),
                                preferred_element_type=jnp.float32)
        diff = q - r
        acc_ref[i] += jnp.sum(diff * diff)
        cnt = jax.lax.dot_general(ones_row, onehot, (((1,), (0,)), ((), ())),
                                  preferred_element_type=jnp.float32)
        counts_ref[pl.ds(i, 1), :] += cnt[0:1, :]
        cur = r + (q - r)
        quant = quant + cur
        r = xb - quant
        idx_out[:, i:i + 1] = idxi
    q_out[...] = quant

    @pl.when(b == NB - 1)
    def _finalize():
        total_quant = jnp.float32(0.0)
        for i in range(CB):
            mloss = acc_ref[i] / jnp.float32(NT * DIM)
            total_quant = total_quant + (mloss + CW * mloss)
        total_util = jnp.float32(0.0)
        for i in range(CB):
            c = counts_ref[pl.ds(i, 1), :]
            total_util = total_util + jnp.mean(jnp.abs(c - jnp.float32(NT / KV)))
        total_compact = jnp.float32(0.0)
        rowi = jax.lax.broadcasted_iota(jnp.int32, (KV, KV), 0)
        coli = jax.lax.broadcasted_iota(jnp.int32, (KV, KV), 1)
        mask = coli > rowi
        for i in range(CB):
            w = w_ref[i]
            sq = jnp.sum(w * w, axis=1)
            m2 = jax.lax.dot_general(w, w, (((1,), (1,)), ((), ())),
                                     preferred_element_type=jnp.float32)
            d2 = (sq[:, None] + sq[None, :]) - 2.0 * m2
            dd = jnp.sqrt(jnp.maximum(d2, 1e-12))
            total_compact = total_compact + 2.0 * (
                jnp.sum(jnp.where(mask, dd, 0.0)) / jnp.float32(NPAIRS))
        lane = jax.lax.broadcasted_iota(jnp.int32, (8, 128), 1)
        sub = jax.lax.broadcasted_iota(jnp.int32, (8, 128), 0)
        out = jnp.where((sub == 0) & (lane == 0), total_quant, 0.0)
        out = jnp.where((sub == 0) & (lane == 1), total_util, out)
        out = jnp.where((sub == 0) & (lane == 2), total_compact, out)
        scal_out[...] = out


def kernel(x, W):
    q, idx3, scal = pl.pallas_call(
        _rvq_body,
        grid=(NB,),
        in_specs=[
            pl.BlockSpec((BT, DIM), lambda b: (b, 0)),
            pl.BlockSpec((CB, KV, DIM), lambda b: (0, 0, 0)),
        ],
        out_specs=[
            pl.BlockSpec((BT, DIM), lambda b: (b, 0)),
            pl.BlockSpec((BT, CB), lambda b: (b, 0)),
            pl.BlockSpec((8, 128), lambda b: (0, 0)),
        ],
        out_shape=[
            jax.ShapeDtypeStruct((NT, DIM), jnp.float32),
            jax.ShapeDtypeStruct((NT, CB), jnp.int32),
            jax.ShapeDtypeStruct((8, 128), jnp.float32),
        ],
        scratch_shapes=[
            pltpu.VMEM((CB, KV), jnp.float32),
            pltpu.SMEM((CB,), jnp.float32),
            pltpu.VMEM((CB, KV), jnp.float32),
        ],
    )(x, W)
    indices = jnp.transpose(idx3, (1, 0))
    return (q, scal[0, 0].reshape(()), scal[0, 1].reshape(()),
            scal[0, 2].reshape(()), indices)
